# baseline (speedup 1.0000x reference)
import functools

import jax
import jax.numpy as jnp
from jax import lax
from jax.experimental import pallas as pl
from jax.experimental.pallas import tpu as pltpu
from jax.experimental.pallas import tpu_sc as plsc

B, L, V, E, O = 4096, 200, 1000, 64, 64
VP = 1024
NK = VP // 128
NC, NS, LANES = 2, 16, 16
NW = NC * NS
NCHUNK = 1
BCH = B // NCHUNK
BPW = BCH // NW
NG = BPW // LANES
LP = L + 1
UNROLL = 8


LT = 208
TP = BPW


def _sc_hist(text_hbm, counts_hbm, rows_v, text_v, hist_a, hist_b, sem_a,
             sem_b):
    c = lax.axis_index("c")
    s = lax.axis_index("s")
    wid = s * NC + c
    base = wid * BPW

    pltpu.sync_copy(text_hbm.at[pl.ds(base, BPW), :], rows_v.at[:, pl.ds(0, L)])

    zeros = jnp.zeros((LANES,), jnp.float32)
    ones = jnp.full((LANES,), 1.0, jnp.float32)
    nones = jnp.full((LANES,), -1.0, jnp.float32)
    rowi = lax.iota(jnp.int32, LANES)

    bufs = (hist_a, hist_b)
    sems = (sem_a, sem_b)

    def zbody(j, carry):
        off = j * LANES
        for ck in range(NK):
            for i in range(LANES):
                hist_a[ck, i, pl.ds(off, LANES)] = zeros
                hist_b[ck, i, pl.ds(off, LANES)] = zeros
        return carry

    lax.fori_loop(0, 128 // LANES, zbody, 0)

    def scatter_pass(hist, col, val):
        def body(o, carry):
            l0 = o * UNROLL
            ids_k = [text_v[pl.ds((l0 + k) * TP + col, LANES)]
                     for k in range(UNROLL)]
            idx_k = [(lax.shift_right_logical(ids, 7),
                      lax.bitwise_and(ids, 127)) for ids in ids_k]
            for hi, lo in idx_k:
                plsc.addupdate_scatter(hist, [hi, rowi, lo], val)
            return carry

        lax.fori_loop(0, L // UNROLL, body, 0)

    rowi_tp = rowi * TP

    def transpose_group(col):
        def tbody(k, carry):
            lvec = rowi_tp + k * (LANES * TP)
            ids_j = [rows_v[col + j, pl.ds(k * LANES, LANES)]
                     for j in range(LANES)]
            idx_j = [lvec + (col + j) for j in range(LANES)]
            for idx, ids in zip(idx_j, ids_j):
                plsc.store_scatter(text_v, [idx], ids)
            return carry

        lax.fori_loop(0, LT // LANES, tbody, 0)

    copies = [None, None]
    for g in range(NG):
        p = g % 2
        hist = bufs[p]
        transpose_group(g * LANES)
        if g >= 2:
            for cp in copies[p]:
                cp.wait()
            scatter_pass(hist, (g - 2) * LANES, nones)
        col = g * LANES
        scatter_pass(hist, col, ones)
        copies[p] = [
            pltpu.async_copy(
                hist.at[k],
                counts_hbm.at[k, pl.ds(base + col, LANES), :],
                sems[p],
            )
            for k in range(NK)
        ]
    for p in range(2):
        for cp in copies[p]:
            cp.wait()


def _tc_matmul(counts_ref, table_ref, w_ref, b_ref, out_ref):
    x = jnp.dot(counts_ref[0], table_ref[0], preferred_element_type=jnp.float32)
    for k in range(1, NK):
        x = x + jnp.dot(counts_ref[k], table_ref[k],
                        preferred_element_type=jnp.float32)
    out = lax.dot_general(x, w_ref[...], (((1,), (1,)), ((), ())),
                          preferred_element_type=jnp.float32)
    out_ref[...] = out * (1.0 / L) + b_ref[...]


BT = 512


def kernel(text, table, W, b):
    mesh = plsc.VectorSubcoreMesh(core_axis_name="c", subcore_axis_name="s")
    hist = functools.partial(
        pl.kernel,
        mesh=mesh,
        out_type=jax.ShapeDtypeStruct((NK, BCH, 128), jnp.float32),
        scratch_types=[
            pltpu.VMEM((BPW, LT), jnp.int32),
            pltpu.VMEM((LT * TP,), jnp.int32),
            pltpu.VMEM((NK, LANES, 128), jnp.float32),
            pltpu.VMEM((NK, LANES, 128), jnp.float32),
            pltpu.SemaphoreType.DMA,
            pltpu.SemaphoreType.DMA,
        ],
        compiler_params=pltpu.CompilerParams(
            use_tc_tiling_on_sc=False, needs_layout_passes=False),
    )(_sc_hist)

    table3 = jnp.pad(table, ((0, VP - V), (0, 0))).reshape(NK, 128, E)
    b2 = b.reshape(1, O)

    mm = pl.pallas_call(
        _tc_matmul,
        grid=(BCH // BT,),
        in_specs=[
            pl.BlockSpec((NK, BT, 128), lambda i: (0, i, 0)),
            pl.BlockSpec((NK, 128, E), lambda i: (0, 0, 0)),
            pl.BlockSpec((O, E), lambda i: (0, 0)),
            pl.BlockSpec((1, O), lambda i: (0, 0)),
        ],
        out_specs=pl.BlockSpec((BT, O), lambda i: (i, 0)),
        out_shape=jax.ShapeDtypeStruct((BCH, O), jnp.float32),
    )

    counts = hist(text)
    return mm(counts, table3, W, b2)


# speedup vs baseline: 1.1805x; 1.1805x over previous
import functools

import jax
import jax.numpy as jnp
from jax import lax
from jax.experimental import pallas as pl
from jax.experimental.pallas import tpu as pltpu
from jax.experimental.pallas import tpu_sc as plsc

B, L, V, E, O = 4096, 200, 1000, 64, 64
VP = 1024
NK = VP // 128
NC, NS, LANES = 2, 16, 16
NW = NC * NS
NCHUNK = 1
BCH = B // NCHUNK
BPW = BCH // NW
NG = BPW // LANES
LP = L + 1
UNROLL = 8


LT = 208
TP = BPW + 1


def _sc_hist(text_hbm, counts_hbm, rows_v, text_v, hist_a, hist_b, sem_a,
             sem_b):
    c = lax.axis_index("c")
    s = lax.axis_index("s")
    wid = s * NC + c
    base = wid * BPW

    pltpu.sync_copy(text_hbm.at[pl.ds(base, BPW), :], rows_v.at[:, pl.ds(0, L)])

    zeros = jnp.zeros((LANES,), jnp.float32)
    ones = jnp.full((LANES,), 1.0, jnp.float32)
    nones = jnp.full((LANES,), -1.0, jnp.float32)
    rowi = lax.iota(jnp.int32, LANES)

    bufs = (hist_a, hist_b)
    sems = (sem_a, sem_b)

    def zbody(j, carry):
        off = j * LANES
        for ck in range(NK):
            for i in range(LANES):
                hist_a[ck, i, pl.ds(off, LANES)] = zeros
                hist_b[ck, i, pl.ds(off, LANES)] = zeros
        return carry

    lax.fori_loop(0, 128 // LANES, zbody, 0)

    def scatter_pass(hist, col, val):
        def body(o, carry):
            l0 = o * UNROLL
            ids_k = [text_v[pl.ds((l0 + k) * TP + col, LANES)]
                     for k in range(UNROLL)]
            idx_k = [(lax.shift_right_logical(ids, 7),
                      lax.bitwise_and(ids, 127)) for ids in ids_k]
            for hi, lo in idx_k:
                plsc.addupdate_scatter(hist, [hi, rowi, lo], val)
            return carry

        lax.fori_loop(0, L // UNROLL, body, 0)

    rowi_tp = rowi * TP

    def transpose_group(col):
        def tbody(k, carry):
            lvec = rowi_tp + k * (LANES * TP)
            ids_j = [rows_v[col + j, pl.ds(k * LANES, LANES)]
                     for j in range(LANES)]
            idx_j = [lvec + (col + j) for j in range(LANES)]
            for idx, ids in zip(idx_j, ids_j):
                plsc.store_scatter(text_v, [idx], ids)
            return carry

        lax.fori_loop(0, LT // LANES, tbody, 0)

    copies = [None, None]
    for g in range(NG):
        p = g % 2
        hist = bufs[p]
        transpose_group(g * LANES)
        if g >= 2:
            for cp in copies[p]:
                cp.wait()
            scatter_pass(hist, (g - 2) * LANES, nones)
        col = g * LANES
        scatter_pass(hist, col, ones)
        copies[p] = [
            pltpu.async_copy(
                hist.at[k],
                counts_hbm.at[k, pl.ds(base + col, LANES), :],
                sems[p],
            )
            for k in range(NK)
        ]
    for p in range(2):
        for cp in copies[p]:
            cp.wait()


def _tc_matmul(counts_ref, table_ref, w_ref, b_ref, out_ref):
    x = jnp.dot(counts_ref[0], table_ref[0], preferred_element_type=jnp.float32)
    for k in range(1, NK):
        x = x + jnp.dot(counts_ref[k], table_ref[k],
                        preferred_element_type=jnp.float32)
    out = lax.dot_general(x, w_ref[...], (((1,), (1,)), ((), ())),
                          preferred_element_type=jnp.float32)
    out_ref[...] = out * (1.0 / L) + b_ref[...]


BT = 512


def kernel(text, table, W, b):
    mesh = plsc.VectorSubcoreMesh(core_axis_name="c", subcore_axis_name="s")
    hist = functools.partial(
        pl.kernel,
        mesh=mesh,
        out_type=jax.ShapeDtypeStruct((NK, BCH, 128), jnp.float32),
        scratch_types=[
            pltpu.VMEM((BPW, LT), jnp.int32),
            pltpu.VMEM((LT * TP,), jnp.int32),
            pltpu.VMEM((NK, LANES, 128), jnp.float32),
            pltpu.VMEM((NK, LANES, 128), jnp.float32),
            pltpu.SemaphoreType.DMA,
            pltpu.SemaphoreType.DMA,
        ],
        compiler_params=pltpu.CompilerParams(
            use_tc_tiling_on_sc=False, needs_layout_passes=False),
    )(_sc_hist)

    table3 = jnp.pad(table, ((0, VP - V), (0, 0))).reshape(NK, 128, E)
    b2 = b.reshape(1, O)

    mm = pl.pallas_call(
        _tc_matmul,
        grid=(BCH // BT,),
        in_specs=[
            pl.BlockSpec((NK, BT, 128), lambda i: (0, i, 0)),
            pl.BlockSpec((NK, 128, E), lambda i: (0, 0, 0)),
            pl.BlockSpec((O, E), lambda i: (0, 0)),
            pl.BlockSpec((1, O), lambda i: (0, 0)),
        ],
        out_specs=pl.BlockSpec((BT, O), lambda i: (i, 0)),
        out_shape=jax.ShapeDtypeStruct((BCH, O), jnp.float32),
    )

    counts = hist(text)
    return mm(counts, table3, W, b2)


# speedup vs baseline: 1.4598x; 1.2366x over previous
import functools

import jax
import jax.numpy as jnp
from jax import lax
from jax.experimental import pallas as pl
from jax.experimental.pallas import tpu as pltpu
from jax.experimental.pallas import tpu_sc as plsc

B, L, V, E, O = 4096, 200, 1000, 64, 64
VP = 1024
NK = VP // 128
NC, NS, LANES = 2, 16, 16
NW = NC * NS
NCHUNK = 1
BCH = B // NCHUNK
BPW = BCH // NW
NG = BPW // LANES
UNROLL = 8


def _sc_hist(textT_hbm, counts_hbm, text_v, hist_a, hist_b, sem_a, sem_b,
             sem_t):
    c = lax.axis_index("c")
    s = lax.axis_index("s")
    wid = s * NC + c
    base = wid * BPW

    stage = pltpu.async_copy(
        textT_hbm.at[:, pl.ds(base, BPW)], text_v, sem_t)

    zeros = jnp.zeros((LANES,), jnp.float32)
    ones = jnp.full((LANES,), 1.0, jnp.float32)
    nones = jnp.full((LANES,), -1.0, jnp.float32)
    rowi = lax.iota(jnp.int32, LANES)

    bufs = (hist_a, hist_b)
    sems = (sem_a, sem_b)

    def zbody(j, carry):
        off = j * LANES
        for ck in range(NK):
            for i in range(LANES):
                hist_a[ck, i, pl.ds(off, LANES)] = zeros
                hist_b[ck, i, pl.ds(off, LANES)] = zeros
        return carry

    lax.fori_loop(0, 128 // LANES, zbody, 0)
    stage.wait()

    def scatter_pass(hist, col, val):
        def body(o, carry):
            l0 = o * UNROLL
            ids_k = [text_v[l0 + k, pl.ds(col, LANES)] for k in range(UNROLL)]
            idx_k = [(lax.shift_right_logical(ids, 7),
                      lax.bitwise_and(ids, 127)) for ids in ids_k]
            for hi, lo in idx_k:
                plsc.addupdate_scatter(hist, [hi, rowi, lo], val)
            return carry

        lax.fori_loop(0, L // UNROLL, body, 0)

    copies = [None, None]
    for g in range(NG):
        p = g % 2
        hist = bufs[p]
        if g >= 2:
            for cp in copies[p]:
                cp.wait()
            scatter_pass(hist, (g - 2) * LANES, nones)
        col = g * LANES
        scatter_pass(hist, col, ones)
        copies[p] = [
            pltpu.async_copy(
                hist.at[k],
                counts_hbm.at[k, pl.ds(base + col, LANES), :],
                sems[p],
            )
            for k in range(NK)
        ]
    for p in range(2):
        for cp in copies[p]:
            cp.wait()


def _tc_matmul(counts_ref, table_ref, w_ref, b_ref, out_ref):
    x = jnp.dot(counts_ref[0], table_ref[0], preferred_element_type=jnp.float32)
    for k in range(1, NK):
        x = x + jnp.dot(counts_ref[k], table_ref[k],
                        preferred_element_type=jnp.float32)
    out = lax.dot_general(x, w_ref[...], (((1,), (1,)), ((), ())),
                          preferred_element_type=jnp.float32)
    out_ref[...] = out * (1.0 / L) + b_ref[...]


BT = 1024


def kernel(text, table, W, b):
    mesh = plsc.VectorSubcoreMesh(core_axis_name="c", subcore_axis_name="s")
    hist = functools.partial(
        pl.kernel,
        mesh=mesh,
        out_type=jax.ShapeDtypeStruct((NK, BCH, 128), jnp.float32),
        scratch_types=[
            pltpu.VMEM((L, BPW), jnp.int32),
            pltpu.VMEM((NK, LANES, 128), jnp.float32),
            pltpu.VMEM((NK, LANES, 128), jnp.float32),
            pltpu.SemaphoreType.DMA,
            pltpu.SemaphoreType.DMA,
            pltpu.SemaphoreType.DMA,
        ],
        compiler_params=pltpu.CompilerParams(
            use_tc_tiling_on_sc=False, needs_layout_passes=False),
    )(_sc_hist)

    table3 = jnp.pad(table, ((0, VP - V), (0, 0))).reshape(NK, 128, E)
    b2 = b.reshape(1, O)

    mm = pl.pallas_call(
        _tc_matmul,
        grid=(BCH // BT,),
        in_specs=[
            pl.BlockSpec((NK, BT, 128), lambda i: (0, i, 0)),
            pl.BlockSpec((NK, 128, E), lambda i: (0, 0, 0)),
            pl.BlockSpec((O, E), lambda i: (0, 0)),
            pl.BlockSpec((1, O), lambda i: (0, 0)),
        ],
        out_specs=pl.BlockSpec((BT, O), lambda i: (i, 0)),
        out_shape=jax.ShapeDtypeStruct((BCH, O), jnp.float32),
    )

    counts = hist(text.T)
    return mm(counts, table3, W, b2)
